# baseline (speedup 1.0000x reference)
import functools

import jax
import jax.numpy as jnp
from jax import lax
from jax.experimental import pallas as pl
from jax.experimental.pallas import tpu as pltpu
from jax.experimental.pallas import tpu_sc as plsc

N = 10000
D = 128
DE = 16
NC, NS = 2, 16
NW = NC * NS
CH = 128
NACC = 10240
NBLK = 1000
NP = 10240
DW = D // 2
EBLK = 2048

_SC_MESH = plsc.VectorSubcoreMesh(
    core_axis_name="c", subcore_axis_name="s", num_cores=NC, num_subcores=NS)


def _node_proj_body(h_ref, w1a_ref, w1b_ref, p_ref, q_ref):
  hb = h_ref[...]
  p_ref[...] = jnp.dot(hb, w1a_ref[...], preferred_element_type=jnp.float32)
  q_ref[...] = jnp.dot(hb, w1b_ref[...], preferred_element_type=jnp.float32)


def _node_proj(h_p, w1a, w1b):
  blk = 1024
  return pl.pallas_call(
      _node_proj_body,
      grid=(NP // blk,),
      in_specs=[
          pl.BlockSpec((blk, D), lambda i: (i, 0)),
          pl.BlockSpec((D, D), lambda i: (0, 0)),
          pl.BlockSpec((D, D), lambda i: (0, 0)),
      ],
      out_specs=(
          pl.BlockSpec((blk, D), lambda i: (i, 0)),
          pl.BlockSpec((blk, D), lambda i: (i, 0)),
      ),
      out_shape=(
          jax.ShapeDtypeStruct((NP, D), jnp.float32),
          jax.ShapeDtypeStruct((NP, D), jnp.float32),
      ),
  )(h_p, w1a, w1b)


def _gather_body(p_hbm, q_hbm, ridx_hbm, cidx_hbm, out_hbm,
                 tspm, idx_b, buf0, buf1, g0, g1, w0, w1, i0, i1):
  nchunk = ridx_hbm.shape[1]
  ew = nchunk * CH
  cid = lax.axis_index("c")
  sid = lax.axis_index("s")
  base = sid * ew
  buf = (buf0, buf1)
  g = (g0, g1)
  w = (w0, w1)
  isem = (i0, i1)
  nrows = p_hbm.shape[0]
  share = nrows // NS
  rbase = sid * share

  def pipeline(tab_hbm, ih, ci):
    pltpu.sync_copy(tab_hbm.at[pl.ds(rbase, share)],
                    tspm.at[pl.ds(rbase, share)])
    plsc.subcore_barrier()

    def start_idx(j, s):
      pltpu.async_copy(ih.at[sid, j], idx_b.at[s], isem[s])

    def wait_idx(j, s):
      pltpu.make_async_copy(ih.at[sid, j], idx_b.at[s], isem[s]).wait()

    def start_gather(s):
      pltpu.async_copy(tspm.at[idx_b.at[s]], buf[s], g[s])

    def wait_gather(s):
      pltpu.make_async_copy(tspm.at[idx_b.at[s]], buf[s], g[s]).wait()

    def start_wb(j, s):
      pltpu.async_copy(buf[s], out_hbm.at[ci, pl.ds(base + j * CH, CH)], w[s])

    def wait_wb(j, s):
      pltpu.make_async_copy(buf[s], out_hbm.at[ci, pl.ds(base + j * CH, CH)],
                            w[s]).wait()

    start_idx(0, 0)
    start_idx(1, 1)
    wait_idx(0, 0)
    start_gather(0)
    wait_gather(0)
    start_wb(0, 0)
    wait_idx(1, 1)
    start_gather(1)
    start_idx(2, 0)
    wait_gather(1)
    wait_wb(0, 0)
    start_wb(1, 1)
    wait_idx(2, 0)
    start_gather(0)
    start_idx(3, 1)

    def steady(gr, carry):
      for b in range(2):
        j = 2 + gr * 2 + b
        s = b
        wait_gather(s)
        wait_wb(j - 1, 1 - s)
        start_wb(j, s)
        wait_idx(j + 1, 1 - s)
        start_gather(1 - s)
        start_idx(j + 2, s)
      return carry

    lax.fori_loop(0, (nchunk - 4) // 2, steady, 0)

    j = nchunk - 2
    s = j % 2
    wait_gather(s)
    wait_wb(j - 1, 1 - s)
    start_wb(j, s)
    wait_idx(j + 1, 1 - s)
    start_gather(1 - s)
    j = nchunk - 1
    s = j % 2
    wait_gather(s)
    wait_wb(j - 1, 1 - s)
    start_wb(j, s)
    wait_wb(j, s)

  @pl.when(cid == 0)
  def _():
    pipeline(p_hbm, ridx_hbm, 0)

  @pl.when(cid == 1)
  def _():
    pipeline(q_hbm, cidx_hbm, 1)


def _gather_pq(p, q, ridx3, cidx3, e_pad):
  nchunk = ridx3.shape[1]
  nrows = p.shape[0]
  return pl.kernel(
      _gather_body,
      out_type=jax.ShapeDtypeStruct((2, e_pad, D), jnp.float32),
      mesh=_SC_MESH,
      scratch_types=(
          [pltpu.VMEM_SHARED((nrows, D), jnp.float32),
           pltpu.VMEM((2, CH), jnp.int32)]
          + [pltpu.VMEM((CH, D), jnp.float32)] * 2
          + [pltpu.SemaphoreType.DMA] * 6
      ),
  )(p, q, ridx3, cidx3)


def _edge_mlp_body(x0_ref, x1_ref, ea_ref, w1c_ref, b1_ref, w2_ref, b2_ref,
                   m_ref):
  t = (x0_ref[0] + x1_ref[0]
       + jnp.dot(ea_ref[...], w1c_ref[...], preferred_element_type=jnp.float32)
       + b1_ref[...])
  t = jnp.maximum(t, 0.0).astype(jnp.bfloat16)
  t = jnp.dot(t, w2_ref[...], preferred_element_type=jnp.float32) + b2_ref[...]
  m_ref[...] = jnp.maximum(t, 0.0)


def _edge_mlp(pq, ea, w1c, b1r, w2, b2r, e_pad):
  return pl.pallas_call(
      _edge_mlp_body,
      grid=(e_pad // EBLK,),
      in_specs=[
          pl.BlockSpec((1, EBLK, D), lambda i: (0, i, 0)),
          pl.BlockSpec((1, EBLK, D), lambda i: (1, i, 0)),
          pl.BlockSpec((EBLK, DE), lambda i: (i, 0)),
          pl.BlockSpec((DE, D), lambda i: (0, 0)),
          pl.BlockSpec((1, D), lambda i: (0, 0)),
          pl.BlockSpec((D, D), lambda i: (0, 0)),
          pl.BlockSpec((1, D), lambda i: (0, 0)),
      ],
      out_specs=pl.BlockSpec((EBLK, D), lambda i: (i, 0)),
      out_shape=jax.ShapeDtypeStruct((e_pad, D), jnp.float32),
  )(pq, pq, ea, w1c, b1r, w2, b2r)


def _scatter_body(m_hbm, sidx_hbm, part_hbm, sidx_v,
                  bm0, bm1, acc, lm0, lm1, ss0, ss1):
  nchunk = sidx_v.shape[0]
  ew = nchunk * CH
  cid = lax.axis_index("c")
  sid = lax.axis_index("s")
  wid = sid * NC + cid
  base = wid * ew
  rows_per_tile = NACC // NS
  bufm = (bm0, bm1)
  lm = (lm0, lm1)
  ss = (ss0, ss1)

  def zrow(i, c2):
    for k in range(D // 16):
      bm0[i, pl.ds(k * 16, 16)] = jnp.zeros((16,), jnp.float32)
    return c2

  lax.fori_loop(0, CH, zrow, 0)
  for t in range(rows_per_tile // CH):
    pltpu.sync_copy(bm0, acc.at[pl.ds(sid * rows_per_tile + t * CH, CH)])
  plsc.subcore_barrier()

  pltpu.sync_copy(sidx_hbm.at[wid], sidx_v)

  def start_load(j, s):
    pltpu.async_copy(m_hbm.at[pl.ds(base + j * CH, CH)], bufm[s], lm[s])

  def wait_load(j, s):
    pltpu.make_async_copy(m_hbm.at[pl.ds(base + j * CH, CH)], bufm[s],
                          lm[s]).wait()

  def start_scatter(j, s):
    pltpu.async_copy(bufm[s], acc.at[sidx_v.at[j]], ss[s], add=True)

  def wait_scatter(j, s):
    pltpu.make_async_copy(bufm[s], acc.at[sidx_v.at[j]], ss[s]).wait()

  start_load(0, 0)
  wait_load(0, 0)
  start_load(1, 1)
  start_scatter(0, 0)

  def steady(g, carry):
    for b in range(2):
      j = 1 + g * 2 + b
      s = (1 + b) % 2
      wait_load(j, s)
      wait_scatter(j - 1, 1 - s)
      start_load(j + 1, 1 - s)
      start_scatter(j, s)
    return carry

  lax.fori_loop(0, (nchunk - 2) // 2, steady, 0)

  jl = nchunk - 1
  wait_load(jl, jl % 2)
  start_scatter(jl, jl % 2)
  wait_scatter(jl - 1, (jl - 1) % 2)
  wait_scatter(jl, jl % 2)

  plsc.subcore_barrier()
  pltpu.sync_copy(acc.at[pl.ds(sid * rows_per_tile, rows_per_tile)],
                  part_hbm.at[cid, pl.ds(sid * rows_per_tile, rows_per_tile)])


def _segment_sum(m, sidx3):
  nchunk = sidx3.shape[1]
  return pl.kernel(
      _scatter_body,
      out_type=jax.ShapeDtypeStruct((NC, NACC, D), jnp.float32),
      mesh=_SC_MESH,
      scratch_types=(
          [pltpu.VMEM((nchunk, CH), jnp.int32)]
          + [pltpu.VMEM((CH, D), jnp.float32)] * 2
          + [pltpu.VMEM_SHARED((NACC, D), jnp.float32)]
          + [pltpu.SemaphoreType.DMA] * 4
      ),
  )(m, sidx3)


def _node_update_body(h_ref, m0_ref, m1_ref, w3_ref, b3_ref, w4_ref, b4_ref,
                      o_ref):
  hb = h_ref[...]
  m = m0_ref[0] + m1_ref[0]
  agg = jnp.concatenate([hb, m], axis=1)
  t = jnp.maximum(
      jnp.dot(agg, w3_ref[...], preferred_element_type=jnp.float32)
      + b3_ref[...], 0.0)
  o_ref[...] = (hb
                + jnp.dot(t, w4_ref[...], preferred_element_type=jnp.float32)
                + b4_ref[...])


def _node_update(h, partials, w3, b3r, w4, b4r):
  return pl.pallas_call(
      _node_update_body,
      grid=(N // NBLK,),
      in_specs=[
          pl.BlockSpec((NBLK, D), lambda i: (i, 0)),
          pl.BlockSpec((1, NBLK, D), lambda i: (0, i, 0)),
          pl.BlockSpec((1, NBLK, D), lambda i: (1, i, 0)),
          pl.BlockSpec((2 * D, D), lambda i: (0, 0)),
          pl.BlockSpec((1, D), lambda i: (0, 0)),
          pl.BlockSpec((D, D), lambda i: (0, 0)),
          pl.BlockSpec((1, D), lambda i: (0, 0)),
      ],
      out_specs=pl.BlockSpec((NBLK, D), lambda i: (i, 0)),
      out_shape=jax.ShapeDtypeStruct((N, D), jnp.float32),
  )(h, partials, partials, w3, b3r, w4, b4r)


def kernel(h, edge_index, edge_attr, W1, b1, W2, b2, W3, b3, W4, b4):
  e = edge_attr.shape[0]
  nchunk = -(-e // (NW * CH))
  nchunk = -(-nchunk // 4) * 4
  e_pad = NW * nchunk * CH
  pad = e_pad - e

  r = edge_index[0].astype(jnp.int32)
  c = edge_index[1].astype(jnp.int32)
  nchunk_t = e_pad // (NS * CH)
  ridx3 = jnp.pad(r, (0, pad)).reshape(NS, nchunk_t, CH)
  cidx3 = jnp.pad(c, (0, pad)).reshape(NS, nchunk_t, CH)
  sidx3 = jnp.pad(r, (0, pad), constant_values=N).reshape(NW, nchunk, CH)
  ea_pad = jnp.pad(edge_attr, ((0, pad), (0, 0)))

  w1a = W1[:D]
  w1b = W1[D:2 * D]
  w1c = W1[2 * D:]
  b1r = b1.reshape(1, D)
  b2r = b2.reshape(1, D)
  b3r = b3.reshape(1, D)
  b4r = b4.reshape(1, D)

  h_p = jnp.pad(h, ((0, NP - N), (0, 0)))
  p, q = _node_proj(h_p, w1a, w1b)
  pq = _gather_pq(p, q, ridx3, cidx3, e_pad)
  m = _edge_mlp(pq, ea_pad, w1c, b1r, W2.astype(jnp.bfloat16), b2r, e_pad)
  partials = _segment_sum(m, sidx3)
  return _node_update(h, partials, W3, b3r, W4, b4r)


# speedup vs baseline: 1.0699x; 1.0699x over previous
import functools

import jax
import jax.numpy as jnp
from jax import lax
from jax.experimental import pallas as pl
from jax.experimental.pallas import tpu as pltpu
from jax.experimental.pallas import tpu_sc as plsc

N = 10000
D = 128
DE = 16
NC, NS = 2, 16
NW = NC * NS
CH = 128
NACC = 10240
NBLK = 1000
NP = 10240
DW = D // 2
EBLK = 2048

_SC_MESH = plsc.VectorSubcoreMesh(
    core_axis_name="c", subcore_axis_name="s", num_cores=NC, num_subcores=NS)


def _node_proj_body(h_ref, w1a_ref, w1b_ref, p_ref, q_ref):
  hb = h_ref[...]
  p_ref[...] = jnp.dot(hb, w1a_ref[...], preferred_element_type=jnp.float32)
  q_ref[...] = jnp.dot(hb, w1b_ref[...], preferred_element_type=jnp.float32)


def _node_proj(h_p, w1a, w1b):
  blk = 1024
  return pl.pallas_call(
      _node_proj_body,
      grid=(NP // blk,),
      in_specs=[
          pl.BlockSpec((blk, D), lambda i: (i, 0)),
          pl.BlockSpec((D, D), lambda i: (0, 0)),
          pl.BlockSpec((D, D), lambda i: (0, 0)),
      ],
      out_specs=(
          pl.BlockSpec((blk, D), lambda i: (i, 0)),
          pl.BlockSpec((blk, D), lambda i: (i, 0)),
      ),
      out_shape=(
          jax.ShapeDtypeStruct((NP, D), jnp.float32),
          jax.ShapeDtypeStruct((NP, D), jnp.float32),
      ),
  )(h_p, w1a, w1b)


def _gather_body(p_hbm, q_hbm, ridx_hbm, cidx_hbm, out_hbm,
                 tspm, idx_b, buf0, buf1, g0, g1, w0, w1, i0, i1):
  nchunk = ridx_hbm.shape[1]
  ew = nchunk * CH
  cid = lax.axis_index("c")
  sid = lax.axis_index("s")
  base = sid * ew
  buf = (buf0, buf1)
  g = (g0, g1)
  w = (w0, w1)
  isem = (i0, i1)
  nrows = p_hbm.shape[0]
  share = nrows // NS
  rbase = sid * share

  def pipeline(tab_hbm, ih, ci):
    pltpu.sync_copy(tab_hbm.at[pl.ds(rbase, share)],
                    tspm.at[pl.ds(rbase, share)])
    plsc.subcore_barrier()

    def start_idx(j, s):
      pltpu.async_copy(ih.at[sid, j], idx_b.at[s], isem[s])

    def wait_idx(j, s):
      pltpu.make_async_copy(ih.at[sid, j], idx_b.at[s], isem[s]).wait()

    def start_gather(s):
      pltpu.async_copy(tspm.at[idx_b.at[s]], buf[s], g[s])

    def wait_gather(s):
      pltpu.make_async_copy(tspm.at[idx_b.at[s]], buf[s], g[s]).wait()

    def start_wb(j, s):
      pltpu.async_copy(buf[s], out_hbm.at[ci, pl.ds(base + j * CH, CH)], w[s])

    def wait_wb(j, s):
      pltpu.make_async_copy(buf[s], out_hbm.at[ci, pl.ds(base + j * CH, CH)],
                            w[s]).wait()

    start_idx(0, 0)
    start_idx(1, 1)
    wait_idx(0, 0)
    start_gather(0)
    wait_gather(0)
    start_wb(0, 0)
    wait_idx(1, 1)
    start_gather(1)
    start_idx(2, 0)
    wait_gather(1)
    wait_wb(0, 0)
    start_wb(1, 1)
    wait_idx(2, 0)
    start_gather(0)
    start_idx(3, 1)

    def steady(gr, carry):
      for b in range(2):
        j = 2 + gr * 2 + b
        s = b
        wait_gather(s)
        wait_wb(j - 1, 1 - s)
        start_wb(j, s)
        wait_idx(j + 1, 1 - s)
        start_gather(1 - s)
        start_idx(j + 2, s)
      return carry

    lax.fori_loop(0, (nchunk - 4) // 2, steady, 0)

    j = nchunk - 2
    s = j % 2
    wait_gather(s)
    wait_wb(j - 1, 1 - s)
    start_wb(j, s)
    wait_idx(j + 1, 1 - s)
    start_gather(1 - s)
    j = nchunk - 1
    s = j % 2
    wait_gather(s)
    wait_wb(j - 1, 1 - s)
    start_wb(j, s)
    wait_wb(j, s)

  @pl.when(cid == 0)
  def _():
    pipeline(p_hbm, ridx_hbm, 0)

  @pl.when(cid == 1)
  def _():
    pipeline(q_hbm, cidx_hbm, 1)


def _gather_pq(p, q, ridx3, cidx3, e_pad):
  nchunk = ridx3.shape[1]
  nrows = p.shape[0]
  return pl.kernel(
      _gather_body,
      out_type=jax.ShapeDtypeStruct((2, e_pad, D), jnp.float32),
      mesh=_SC_MESH,
      scratch_types=(
          [pltpu.VMEM_SHARED((nrows, D), jnp.float32),
           pltpu.VMEM((2, CH), jnp.int32)]
          + [pltpu.VMEM((CH, D), jnp.float32)] * 2
          + [pltpu.SemaphoreType.DMA] * 6
      ),
  )(p, q, ridx3, cidx3)


def _edge_mlp_body(x0_ref, x1_ref, ea_ref, w1c_ref, b1_ref, w2_ref, b2_ref,
                   m_ref):
  t = (x0_ref[0] + x1_ref[0]
       + jnp.dot(ea_ref[...], w1c_ref[...], preferred_element_type=jnp.float32)
       + b1_ref[...])
  t = jnp.maximum(t, 0.0).astype(jnp.bfloat16)
  t = jnp.dot(t, w2_ref[...], preferred_element_type=jnp.float32) + b2_ref[...]
  m_ref[...] = jnp.maximum(t, 0.0)


def _edge_mlp(pq, ea, w1c, b1r, w2, b2r, e_pad):
  return pl.pallas_call(
      _edge_mlp_body,
      grid=(e_pad // EBLK,),
      in_specs=[
          pl.BlockSpec((1, EBLK, D), lambda i: (0, i, 0)),
          pl.BlockSpec((1, EBLK, D), lambda i: (1, i, 0)),
          pl.BlockSpec((EBLK, DE), lambda i: (i, 0)),
          pl.BlockSpec((DE, D), lambda i: (0, 0)),
          pl.BlockSpec((1, D), lambda i: (0, 0)),
          pl.BlockSpec((D, D), lambda i: (0, 0)),
          pl.BlockSpec((1, D), lambda i: (0, 0)),
      ],
      out_specs=pl.BlockSpec((EBLK, D), lambda i: (i, 0)),
      out_shape=jax.ShapeDtypeStruct((e_pad, D), jnp.float32),
  )(pq, pq, ea, w1c, b1r, w2, b2r)


def _scatter_body(m_hbm, sidx_hbm, part_hbm, sidx_v,
                  bm0, bm1, acc, lm0, lm1, ss0, ss1):
  nchunk = sidx_v.shape[0]
  ew = nchunk * CH
  cid = lax.axis_index("c")
  sid = lax.axis_index("s")
  wid = sid * NC + cid
  base = wid * ew
  rows_per_tile = NACC // NS
  bufm = (bm0, bm1)
  lm = (lm0, lm1)
  ss = (ss0, ss1)

  def zrow(i, c2):
    for k in range(D // 16):
      bm0[i, pl.ds(k * 16, 16)] = jnp.zeros((16,), jnp.float32)
    return c2

  lax.fori_loop(0, CH, zrow, 0)
  for t in range(rows_per_tile // CH):
    pltpu.sync_copy(bm0, acc.at[pl.ds(sid * rows_per_tile + t * CH, CH)])
  plsc.subcore_barrier()

  pltpu.sync_copy(sidx_hbm.at[wid], sidx_v)

  def start_load(j, s):
    pltpu.async_copy(m_hbm.at[pl.ds(base + j * CH, CH)], bufm[s], lm[s])

  def wait_load(j, s):
    pltpu.make_async_copy(m_hbm.at[pl.ds(base + j * CH, CH)], bufm[s],
                          lm[s]).wait()

  def start_scatter(j, s):
    pltpu.async_copy(bufm[s], acc.at[sidx_v.at[j]], ss[s], add=True)

  def wait_scatter(j, s):
    pltpu.make_async_copy(bufm[s], acc.at[sidx_v.at[j]], ss[s]).wait()

  start_load(0, 0)
  wait_load(0, 0)
  start_load(1, 1)
  start_scatter(0, 0)

  def steady(g, carry):
    for b in range(2):
      j = 1 + g * 2 + b
      s = (1 + b) % 2
      wait_load(j, s)
      wait_scatter(j - 1, 1 - s)
      start_load(j + 1, 1 - s)
      start_scatter(j, s)
    return carry

  lax.fori_loop(0, (nchunk - 2) // 2, steady, 0)

  jl = nchunk - 1
  wait_load(jl, jl % 2)
  start_scatter(jl, jl % 2)
  wait_scatter(jl - 1, (jl - 1) % 2)
  wait_scatter(jl, jl % 2)

  plsc.subcore_barrier()
  pltpu.sync_copy(acc.at[pl.ds(sid * rows_per_tile, rows_per_tile)],
                  part_hbm.at[cid, pl.ds(sid * rows_per_tile, rows_per_tile)])


def _segment_sum(m, sidx3):
  nchunk = sidx3.shape[1]
  return pl.kernel(
      _scatter_body,
      out_type=jax.ShapeDtypeStruct((NC, NACC, D), jnp.float32),
      mesh=_SC_MESH,
      scratch_types=(
          [pltpu.VMEM((nchunk, CH), jnp.int32)]
          + [pltpu.VMEM((CH, D), jnp.float32)] * 2
          + [pltpu.VMEM_SHARED((NACC, D), jnp.float32)]
          + [pltpu.SemaphoreType.DMA] * 4
      ),
  )(m, sidx3)


def _node_update_body(h_ref, *rest):
  m_refs = rest[:-5]
  w3_ref, b3_ref, w4_ref, b4_ref, o_ref = rest[-5:]
  hb = h_ref[...]
  m = m_refs[0][0]
  for mr in m_refs[1:]:
    m = m + mr[0]
  agg = jnp.concatenate([hb, m], axis=1)
  t = jnp.maximum(
      jnp.dot(agg, w3_ref[...], preferred_element_type=jnp.float32)
      + b3_ref[...], 0.0)
  o_ref[...] = (hb
                + jnp.dot(t, w4_ref[...], preferred_element_type=jnp.float32)
                + b4_ref[...])


def _node_update(h, partials_list, w3, b3r, w4, b4r):
  m_args = []
  m_specs = []
  for pa in partials_list:
    for cplane in range(NC):
      m_args.append(pa)
      m_specs.append(
          pl.BlockSpec((1, NBLK, D), lambda i, _c=cplane: (_c, i, 0)))
  return pl.pallas_call(
      _node_update_body,
      grid=(N // NBLK,),
      in_specs=([pl.BlockSpec((NBLK, D), lambda i: (i, 0))]
                + m_specs
                + [
                    pl.BlockSpec((2 * D, D), lambda i: (0, 0)),
                    pl.BlockSpec((1, D), lambda i: (0, 0)),
                    pl.BlockSpec((D, D), lambda i: (0, 0)),
                    pl.BlockSpec((1, D), lambda i: (0, 0)),
                ]),
      out_specs=pl.BlockSpec((NBLK, D), lambda i: (i, 0)),
      out_shape=jax.ShapeDtypeStruct((N, D), jnp.float32),
  )(h, *m_args, w3, b3r, w4, b4r)


NSLICE = 2


def kernel(h, edge_index, edge_attr, W1, b1, W2, b2, W3, b3, W4, b4):
  e = edge_attr.shape[0]
  align = NW * CH * NSLICE
  e_pad = -(-e // align) * align
  es = e_pad // NSLICE
  assert es % (NS * CH) == 0 and es % (NW * CH) == 0
  nch_g = es // (NS * CH)
  nch_s = es // (NW * CH)
  assert nch_g % 2 == 0 and nch_g >= 4 and nch_s % 2 == 0 and nch_s >= 2
  pad = e_pad - e

  r = edge_index[0].astype(jnp.int32)
  c = edge_index[1].astype(jnp.int32)
  r_pad = jnp.pad(r, (0, pad))
  c_pad = jnp.pad(c, (0, pad))
  s_pad = jnp.pad(r, (0, pad), constant_values=N)
  ea_pad = jnp.pad(edge_attr, ((0, pad), (0, 0)))

  w1a = W1[:D]
  w1b = W1[D:2 * D]
  w1c = W1[2 * D:]
  w2b = W2.astype(jnp.bfloat16)
  b1r = b1.reshape(1, D)
  b2r = b2.reshape(1, D)
  b3r = b3.reshape(1, D)
  b4r = b4.reshape(1, D)

  h_p = jnp.pad(h, ((0, NP - N), (0, 0)))
  p, q = _node_proj(h_p, w1a, w1b)

  partials_list = []
  for k in range(NSLICE):
    sl = slice(k * es, (k + 1) * es)
    ridx3 = r_pad[sl].reshape(NS, nch_g, CH)
    cidx3 = c_pad[sl].reshape(NS, nch_g, CH)
    sidx3 = s_pad[sl].reshape(NW, nch_s, CH)
    pq = _gather_pq(p, q, ridx3, cidx3, es)
    m = _edge_mlp(pq, ea_pad[sl], w1c, b1r, w2b, b2r, es)
    partials_list.append(_segment_sum(m, sidx3))
  return _node_update(h, partials_list, W3, b3r, W4, b4r)
